# baseline (speedup 1.0000x reference)
import functools

import jax
import jax.numpy as jnp
from jax import lax
from jax.experimental import pallas as pl
from jax.experimental.pallas import tpu as pltpu
from jax.experimental.pallas import tpu_sc as plsc

EMBED_DIM = 384
MAX_POS = 64
D3 = EMBED_DIM // 3

NUM_CORES = 2
NUM_SUBCORES = 16
NUM_WORKERS = NUM_CORES * NUM_SUBCORES

TOKENS = 4 * 16384
TOK_PER_W = TOKENS // NUM_WORKERS
CHUNK = 64
NCHUNK = TOK_PER_W // CHUNK
GROUPS = CHUNK // 16
TBL_ROWS = 3 * MAX_POS


def _project_tables_body(d_ref, h_ref, w_ref, pwt_ref, pb_ref, out_ref):
    bias = pb_ref[0, :]
    td = jnp.dot(d_ref[...], pwt_ref[0:D3, :], preferred_element_type=jnp.float32)
    th = jnp.dot(h_ref[...], pwt_ref[D3:2 * D3, :], preferred_element_type=jnp.float32)
    tw = jnp.dot(w_ref[...], pwt_ref[2 * D3:3 * D3, :], preferred_element_type=jnp.float32)
    out_ref[0:MAX_POS, :] = td + bias[None, :]
    out_ref[MAX_POS:2 * MAX_POS, :] = th
    out_ref[2 * MAX_POS:3 * MAX_POS, :] = tw


def _project_tables(d_table, h_table, w_table, proj_w, proj_b):
    return pl.pallas_call(
        _project_tables_body,
        out_shape=jax.ShapeDtypeStruct((TBL_ROWS, EMBED_DIM), jnp.float32),
    )(d_table, h_table, w_table, proj_w.T, proj_b.reshape(1, EMBED_DIM))


def _sc_body(pos_hbm, tc_hbm, out_hbm, posv, tcv, ob0, ob1, sem0, sem1):
    wid = lax.axis_index("s") * NUM_CORES + lax.axis_index("c")
    base = wid * TOK_PER_W

    pltpu.sync_copy(pos_hbm.at[pl.ds(base * 3, TOK_PER_W * 3)], posv)
    pltpu.sync_copy(tc_hbm, tcv)

    iota16 = lax.iota(jnp.int32, 16)
    obufs = (ob0, ob1)
    sems = (sem0, sem1)

    def process_group(c, g, obuf):
        tloc3 = (jnp.full((16,), c * CHUNK + g * 16, jnp.int32) + iota16) * 3
        pd = plsc.load_gather(posv, [tloc3])
        ph = plsc.load_gather(posv, [tloc3 + 1])
        pw = plsc.load_gather(posv, [tloc3 + 2])
        rd = jnp.clip(pd, 0, MAX_POS - 1) * EMBED_DIM
        rh = (jnp.clip(ph, 0, MAX_POS - 1) + MAX_POS) * EMBED_DIM
        rw = (jnp.clip(pw, 0, MAX_POS - 1) + 2 * MAX_POS) * EMBED_DIM
        lg = (g * 16 + iota16) * EMBED_DIM

        def jbody(j, carry):
            vd = plsc.load_gather(tcv, [rd + j])
            vh = plsc.load_gather(tcv, [rh + j])
            vw = plsc.load_gather(tcv, [rw + j])
            plsc.store_scatter(obuf, [lg + j], vd + vh + vw)
            return carry

        lax.fori_loop(0, EMBED_DIM, jbody, 0, unroll=8)

    def chunk_pair(cp, carry):
        for b in range(2):
            c = cp * 2 + b

            @pl.when(cp > 0)
            def _wait_prev():
                pltpu.make_async_copy(
                    obufs[b], out_hbm.at[pl.ds(base * EMBED_DIM, CHUNK * EMBED_DIM)],
                    sems[b],
                ).wait()

            for g in range(GROUPS):
                process_group(c, g, obufs[b])

            pltpu.make_async_copy(
                obufs[b],
                out_hbm.at[pl.ds((base + c * CHUNK) * EMBED_DIM, CHUNK * EMBED_DIM)],
                sems[b],
            ).start()
        return carry

    lax.fori_loop(0, NCHUNK // 2, chunk_pair, 0)

    for b in range(2):
        pltpu.make_async_copy(
            obufs[b], out_hbm.at[pl.ds(base * EMBED_DIM, CHUNK * EMBED_DIM)], sems[b]
        ).wait()


_sc_gather = functools.partial(
    pl.kernel,
    out_type=jax.ShapeDtypeStruct((TOKENS * EMBED_DIM,), jnp.float32),
    mesh=plsc.VectorSubcoreMesh(
        core_axis_name="c", subcore_axis_name="s",
        num_cores=NUM_CORES, num_subcores=NUM_SUBCORES,
    ),
    compiler_params=pltpu.CompilerParams(needs_layout_passes=False),
    scratch_types=[
        pltpu.VMEM((TOK_PER_W * 3,), jnp.int32),
        pltpu.VMEM((TBL_ROWS * EMBED_DIM,), jnp.float32),
        pltpu.VMEM((CHUNK * EMBED_DIM,), jnp.float32),
        pltpu.VMEM((CHUNK * EMBED_DIM,), jnp.float32),
        pltpu.SemaphoreType.DMA,
        pltpu.SemaphoreType.DMA,
    ],
)(_sc_body)


@jax.jit
def kernel(positions, d_table, h_table, w_table, proj_w, proj_b):
    tc = _project_tables(d_table, h_table, w_table, proj_w, proj_b)
    pos = positions.astype(jnp.int32).reshape(TOKENS * 3)
    out = _sc_gather(pos, tc.reshape(TBL_ROWS * EMBED_DIM))
    return out.reshape(positions.shape[0], positions.shape[1], EMBED_DIM)


# speedup vs baseline: 2.3937x; 2.3937x over previous
import functools

import jax
import jax.numpy as jnp
from jax import lax
from jax.experimental import pallas as pl
from jax.experimental.pallas import tpu as pltpu
from jax.experimental.pallas import tpu_sc as plsc

EMBED_DIM = 384
MAX_POS = 64
D3 = EMBED_DIM // 3

NUM_CORES = 2
NUM_SUBCORES = 16
NUM_WORKERS = NUM_CORES * NUM_SUBCORES

TOKENS = 4 * 16384
TOK_PER_W = TOKENS // NUM_WORKERS
CHUNK = 64
NCHUNK = TOK_PER_W // CHUNK
GROUPS = CHUNK // 16
TBL_ROWS = 3 * MAX_POS
TBL_STRIDE = EMBED_DIM + 1


def _project_tables_body(d_ref, h_ref, w_ref, pwt_ref, pb_ref, out_ref):
    bias = pb_ref[0, :]
    td = jnp.dot(d_ref[...], pwt_ref[0:D3, :], preferred_element_type=jnp.float32)
    th = jnp.dot(h_ref[...], pwt_ref[D3:2 * D3, :], preferred_element_type=jnp.float32)
    tw = jnp.dot(w_ref[...], pwt_ref[2 * D3:3 * D3, :], preferred_element_type=jnp.float32)
    out_ref[0:MAX_POS, 0:EMBED_DIM] = td + bias[None, :]
    out_ref[MAX_POS:2 * MAX_POS, 0:EMBED_DIM] = th
    out_ref[2 * MAX_POS:3 * MAX_POS, 0:EMBED_DIM] = tw
    out_ref[:, EMBED_DIM:TBL_STRIDE] = jnp.zeros((TBL_ROWS, TBL_STRIDE - EMBED_DIM), jnp.float32)


def _project_tables(d_table, h_table, w_table, proj_w, proj_b):
    return pl.pallas_call(
        _project_tables_body,
        out_shape=jax.ShapeDtypeStruct((TBL_ROWS, TBL_STRIDE), jnp.float32),
    )(d_table, h_table, w_table, proj_w.T, proj_b.reshape(1, EMBED_DIM))


def _sc_body(pos_hbm, tc_hbm, out_hbm, posv, tcv, ob0, ob1, sem0, sem1):
    wid = lax.axis_index("s") * NUM_CORES + lax.axis_index("c")
    base = wid * TOK_PER_W

    pltpu.sync_copy(pos_hbm.at[pl.ds(base * 3, TOK_PER_W * 3)], posv)
    pltpu.sync_copy(tc_hbm, tcv)

    iota16 = lax.iota(jnp.int32, 16)
    obufs = (ob0, ob1)
    sems = (sem0, sem1)

    def process_group(c, g, obuf):
        tloc3 = (jnp.full((16,), c * CHUNK + g * 16, jnp.int32) + iota16) * 3
        pd = plsc.load_gather(posv, [tloc3])
        ph = plsc.load_gather(posv, [tloc3 + 1])
        pw = plsc.load_gather(posv, [tloc3 + 2])
        rd = jnp.clip(pd, 0, MAX_POS - 1) * TBL_STRIDE
        rh = (jnp.clip(ph, 0, MAX_POS - 1) + MAX_POS) * TBL_STRIDE
        rw = (jnp.clip(pw, 0, MAX_POS - 1) + 2 * MAX_POS) * TBL_STRIDE
        lg = (g * 16 + iota16) * EMBED_DIM

        def jbody(j, carry):
            vd = plsc.load_gather(tcv, [rd + j])
            vh = plsc.load_gather(tcv, [rh + j])
            vw = plsc.load_gather(tcv, [rw + j])
            plsc.store_scatter(obuf, [lg + j], vd + vh + vw)
            return carry

        lax.fori_loop(0, EMBED_DIM, jbody, 0, unroll=8)

    def chunk_pair(cp, carry):
        for b in range(2):
            c = cp * 2 + b

            @pl.when(cp > 0)
            def _wait_prev():
                pltpu.make_async_copy(
                    obufs[b], out_hbm.at[pl.ds(base * EMBED_DIM, CHUNK * EMBED_DIM)],
                    sems[b],
                ).wait()

            for g in range(GROUPS):
                process_group(c, g, obufs[b])

            pltpu.make_async_copy(
                obufs[b],
                out_hbm.at[pl.ds((base + c * CHUNK) * EMBED_DIM, CHUNK * EMBED_DIM)],
                sems[b],
            ).start()
        return carry

    lax.fori_loop(0, NCHUNK // 2, chunk_pair, 0)

    for b in range(2):
        pltpu.make_async_copy(
            obufs[b], out_hbm.at[pl.ds(base * EMBED_DIM, CHUNK * EMBED_DIM)], sems[b]
        ).wait()


_sc_gather = functools.partial(
    pl.kernel,
    out_type=jax.ShapeDtypeStruct((TOKENS * EMBED_DIM,), jnp.float32),
    mesh=plsc.VectorSubcoreMesh(
        core_axis_name="c", subcore_axis_name="s",
        num_cores=NUM_CORES, num_subcores=NUM_SUBCORES,
    ),
    compiler_params=pltpu.CompilerParams(needs_layout_passes=False),
    scratch_types=[
        pltpu.VMEM((TOK_PER_W * 3,), jnp.int32),
        pltpu.VMEM((TBL_ROWS * TBL_STRIDE,), jnp.float32),
        pltpu.VMEM((CHUNK * EMBED_DIM,), jnp.float32),
        pltpu.VMEM((CHUNK * EMBED_DIM,), jnp.float32),
        pltpu.SemaphoreType.DMA,
        pltpu.SemaphoreType.DMA,
    ],
)(_sc_body)


@jax.jit
def kernel(positions, d_table, h_table, w_table, proj_w, proj_b):
    tc = _project_tables(d_table, h_table, w_table, proj_w, proj_b)
    pos = positions.astype(jnp.int32).reshape(TOKENS * 3)
    out = _sc_gather(pos, tc.reshape(TBL_ROWS * TBL_STRIDE))
    return out.reshape(positions.shape[0], positions.shape[1], EMBED_DIM)
